# initial kernel scaffold (unmeasured)
import jax
import jax.numpy as jnp
from jax import lax
from jax.experimental import pallas as pl
from jax.experimental.pallas import tpu as pltpu

N_DEV = 4
B, S, D = 2, 512, 2048
H, DH, DR = 16, 128, 32
DC = 512
DCS = DC // N_DEV
BS = B * S
N_COMM = 3


def _gather_body(x_ref, wdkv_ref, wuk_ref, wuv_ref, wkr_ref,
                 c_ref, wukf_ref, wuvf_ref, kr_ref,
                 send_sems, recv_sems):
    my = lax.axis_index("i")
    right = lax.rem(my + 1, N_DEV)

    x2d = x_ref[...].reshape(BS, D)
    c_ref[my] = jnp.dot(x2d, wdkv_ref[...], preferred_element_type=jnp.float32)
    wukf_ref[my] = wuk_ref[...]
    wuvf_ref[my] = wuv_ref[...]
    kr_ref[...] = jnp.dot(x2d, wkr_ref[...], preferred_element_type=jnp.float32)

    for h in range(N_DEV - 1):
        o = lax.rem(my + (N_DEV - h), N_DEV)
        rdmas = []
        for t, ref in enumerate((c_ref, wukf_ref, wuvf_ref)):
            rdma = pltpu.make_async_remote_copy(
                src_ref=ref.at[o],
                dst_ref=ref.at[o],
                send_sem=send_sems.at[h, t],
                recv_sem=recv_sems.at[h, t],
                device_id=(right,),
                device_id_type=pl.DeviceIdType.MESH,
            )
            rdma.start()
            rdmas.append(rdma)
        for rdma in rdmas:
            rdma.wait()


def _mla_body(x_ref, c_ref, wukf_ref, wuvf_ref, kr_ref,
              wq_ref, wqr_ref, wo_ref, out_ref):
    h = pl.program_id(0)
    scale = (DH + DR) ** -0.5

    x2d = x_ref[...].reshape(BS, D)
    q = jnp.dot(x2d, wq_ref[...], preferred_element_type=jnp.float32)
    qr = jnp.dot(x2d, wqr_ref[...], preferred_element_type=jnp.float32)

    k = jnp.zeros((BS, DH), jnp.float32)
    v = jnp.zeros((BS, DH), jnp.float32)
    for o in range(N_DEV):
        k = k + jnp.dot(c_ref[o], wukf_ref[o], preferred_element_type=jnp.float32)
        v = v + jnp.dot(c_ref[o], wuvf_ref[o], preferred_element_type=jnp.float32)
    kr = kr_ref[...]

    outs = []
    nt = (((1,), (1,)), ((), ()))
    for b in range(B):
        sl = slice(b * S, (b + 1) * S)
        s = lax.dot_general(q[sl], k[sl], nt, preferred_element_type=jnp.float32)
        s = s + lax.dot_general(qr[sl], kr[sl], nt,
                                preferred_element_type=jnp.float32)
        s = s * scale
        m = jnp.max(s, axis=-1, keepdims=True)
        p = jnp.exp(s - m)
        p = p / jnp.sum(p, axis=-1, keepdims=True)
        outs.append(jnp.dot(p, v[sl], preferred_element_type=jnp.float32))
    o_h = jnp.concatenate(outs, axis=0)

    contrib = jnp.dot(o_h, wo_ref[...],
                      preferred_element_type=jnp.float32).reshape(B, S, D)

    @pl.when(h == 0)
    def _():
        out_ref[...] = contrib

    @pl.when(h != 0)
    def _():
        out_ref[...] = out_ref[...] + contrib


def kernel(x, Wdkv, Wuk, Wuv, Wq, Wqr, Wkr, Wo):
    c, wukf, wuvf, kr = pl.pallas_call(
        _gather_body,
        out_shape=(
            jax.ShapeDtypeStruct((N_DEV, BS, DCS), jnp.float32),
            jax.ShapeDtypeStruct((N_DEV, DCS, D), jnp.float32),
            jax.ShapeDtypeStruct((N_DEV, DCS, D), jnp.float32),
            jax.ShapeDtypeStruct((BS, DR), jnp.float32),
        ),
        in_specs=[pl.BlockSpec(memory_space=pltpu.VMEM)] * 5,
        out_specs=(pl.BlockSpec(memory_space=pltpu.VMEM),) * 4,
        scratch_shapes=[
            pltpu.SemaphoreType.DMA((N_DEV - 1, N_COMM)),
            pltpu.SemaphoreType.DMA((N_DEV - 1, N_COMM)),
        ],
    )(x, Wdkv, Wuk, Wuv, Wkr)

    out = pl.pallas_call(
        _mla_body,
        grid=(H,),
        out_shape=jax.ShapeDtypeStruct((B, S, D), jnp.float32),
        in_specs=[
            pl.BlockSpec((B, S, D), lambda h: (0, 0, 0)),
            pl.BlockSpec((N_DEV, BS, DCS), lambda h: (0, 0, 0)),
            pl.BlockSpec((N_DEV, DCS, DH), lambda h: (0, 0, h)),
            pl.BlockSpec((N_DEV, DCS, DH), lambda h: (0, 0, h)),
            pl.BlockSpec((BS, DR), lambda h: (0, 0)),
            pl.BlockSpec((D, DH), lambda h: (0, h)),
            pl.BlockSpec((D, DR), lambda h: (0, h)),
            pl.BlockSpec((DH, D), lambda h: (h, 0)),
        ],
        out_specs=pl.BlockSpec((B, S, D), lambda h: (0, 0, 0)),
        compiler_params=pltpu.CompilerParams(
            dimension_semantics=("arbitrary",),
        ),
    )(x, c, wukf, wuvf, kr, Wq, Wqr, Wo)
    return out


# baseline (device time: 218040 ns/iter reference)
import jax
import jax.numpy as jnp
from jax import lax
from jax.experimental import pallas as pl
from jax.experimental.pallas import tpu as pltpu

N_DEV = 4
B, S, D = 2, 512, 2048
H, DH, DR = 16, 128, 32
DC = 512
DCS = DC // N_DEV
BS = B * S
N_COMM = 3


def _gather_body(x_ref, wdkv_ref, wuk_ref, wuv_ref, wkr_ref, wqr_ref,
                 c_ref, wukf_ref, wuvf_ref, kr_ref, qr_ref,
                 send_sems, recv_sems):
    my = lax.axis_index("i")
    right = lax.rem(my + 1, N_DEV)

    x2d = x_ref[...].reshape(BS, D)
    c_ref[my] = jnp.dot(x2d, wdkv_ref[...], preferred_element_type=jnp.float32)
    wukf_ref[my] = wuk_ref[...]
    wuvf_ref[my] = wuv_ref[...]
    kr_ref[...] = jnp.dot(x2d, wkr_ref[...], preferred_element_type=jnp.float32)
    qr_full = jnp.dot(x2d, wqr_ref[...], preferred_element_type=jnp.float32)
    for hh in range(H):
        qr_ref[hh] = qr_full[:, hh * DR:(hh + 1) * DR]

    for h in range(N_DEV - 1):
        o = lax.rem(my + (N_DEV - h), N_DEV)
        rdmas = []
        for t, ref in enumerate((c_ref, wukf_ref, wuvf_ref)):
            rdma = pltpu.make_async_remote_copy(
                src_ref=ref.at[o],
                dst_ref=ref.at[o],
                send_sem=send_sems.at[h, t],
                recv_sem=recv_sems.at[h, t],
                device_id=(right,),
                device_id_type=pl.DeviceIdType.MESH,
            )
            rdma.start()
            rdmas.append(rdma)
        for rdma in rdmas:
            rdma.wait()


def _mla_body(x_ref, c_ref, wukf_ref, wuvf_ref, kr_ref, qr_h_ref,
              wq_ref, wo_ref, out_ref):
    h = pl.program_id(0)
    scale = (DH + DR) ** -0.5

    x2d = x_ref[...].reshape(BS, D)
    q = jnp.dot(x2d, wq_ref[...], preferred_element_type=jnp.float32)
    qr = qr_h_ref[0]

    k = jnp.zeros((BS, DH), jnp.float32)
    v = jnp.zeros((BS, DH), jnp.float32)
    for o in range(N_DEV):
        k = k + jnp.dot(c_ref[o], wukf_ref[o], preferred_element_type=jnp.float32)
        v = v + jnp.dot(c_ref[o], wuvf_ref[o], preferred_element_type=jnp.float32)
    kr = kr_ref[...]

    outs = []
    nt = (((1,), (1,)), ((), ()))
    for b in range(B):
        sl = slice(b * S, (b + 1) * S)
        s = lax.dot_general(q[sl], k[sl], nt, preferred_element_type=jnp.float32)
        s = s + lax.dot_general(qr[sl], kr[sl], nt,
                                preferred_element_type=jnp.float32)
        s = s * scale
        m = jnp.max(s, axis=-1, keepdims=True)
        p = jnp.exp(s - m)
        p = p / jnp.sum(p, axis=-1, keepdims=True)
        outs.append(jnp.dot(p, v[sl], preferred_element_type=jnp.float32))
    o_h = jnp.concatenate(outs, axis=0)

    contrib = jnp.dot(o_h, wo_ref[...],
                      preferred_element_type=jnp.float32).reshape(B, S, D)

    @pl.when(h == 0)
    def _():
        out_ref[...] = contrib

    @pl.when(h != 0)
    def _():
        out_ref[...] = out_ref[...] + contrib


def kernel(x, Wdkv, Wuk, Wuv, Wq, Wqr, Wkr, Wo):
    c, wukf, wuvf, kr, qr = pl.pallas_call(
        _gather_body,
        out_shape=(
            jax.ShapeDtypeStruct((N_DEV, BS, DCS), jnp.float32),
            jax.ShapeDtypeStruct((N_DEV, DCS, D), jnp.float32),
            jax.ShapeDtypeStruct((N_DEV, DCS, D), jnp.float32),
            jax.ShapeDtypeStruct((BS, DR), jnp.float32),
            jax.ShapeDtypeStruct((H, BS, DR), jnp.float32),
        ),
        in_specs=[pl.BlockSpec(memory_space=pltpu.VMEM)] * 6,
        out_specs=(pl.BlockSpec(memory_space=pltpu.VMEM),) * 5,
        scratch_shapes=[
            pltpu.SemaphoreType.DMA((N_DEV - 1, N_COMM)),
            pltpu.SemaphoreType.DMA((N_DEV - 1, N_COMM)),
        ],
    )(x, Wdkv, Wuk, Wuv, Wkr, Wqr)

    out = pl.pallas_call(
        _mla_body,
        grid=(H,),
        out_shape=jax.ShapeDtypeStruct((B, S, D), jnp.float32),
        in_specs=[
            pl.BlockSpec((B, S, D), lambda h: (0, 0, 0)),
            pl.BlockSpec((N_DEV, BS, DCS), lambda h: (0, 0, 0)),
            pl.BlockSpec((N_DEV, DCS, DH), lambda h: (0, 0, h)),
            pl.BlockSpec((N_DEV, DCS, DH), lambda h: (0, 0, h)),
            pl.BlockSpec((BS, DR), lambda h: (0, 0)),
            pl.BlockSpec((1, BS, DR), lambda h: (h, 0, 0)),
            pl.BlockSpec((D, DH), lambda h: (0, h)),
            pl.BlockSpec((DH, D), lambda h: (h, 0)),
        ],
        out_specs=pl.BlockSpec((B, S, D), lambda h: (0, 0, 0)),
        compiler_params=pltpu.CompilerParams(
            dimension_semantics=("arbitrary",),
        ),
    )(x, c, wukf, wuvf, kr, qr, Wq, Wo)
    return out


# device time: 157349 ns/iter; 1.3857x vs baseline; 1.3857x over previous
import jax
import jax.numpy as jnp
from jax import lax
from jax.experimental import pallas as pl
from jax.experimental.pallas import tpu as pltpu

N_DEV = 4
B, S, D = 2, 512, 2048
H, DH, DR = 16, 128, 32
DC = 512
DCS = DC // N_DEV
BS = B * S
N_COMM = 3
N_PEER = N_DEV - 1


def _gather_body(x_ref, wdkv_ref, wuk_ref, wuv_ref, wkr_ref, wqr_ref,
                 c_ref, wukf_ref, wuvf_ref, kr_ref, qr_ref, xbf_ref,
                 send_sems, recv_sems):
    my = lax.axis_index("i")

    x2d = x_ref[...].reshape(BS, D)
    xbf = x2d.astype(jnp.bfloat16)
    xbf_ref[...] = xbf
    c_ref[my] = jnp.dot(
        xbf, wdkv_ref[...].astype(jnp.bfloat16),
        preferred_element_type=jnp.float32).astype(jnp.bfloat16)
    wukf_ref[my] = wuk_ref[...].astype(jnp.bfloat16)
    wuvf_ref[my] = wuv_ref[...].astype(jnp.bfloat16)

    rdmas = []
    for p in range(1, N_DEV):
        dst = lax.rem(my + p, N_DEV)
        for t, ref in enumerate((c_ref, wukf_ref, wuvf_ref)):
            rdma = pltpu.make_async_remote_copy(
                src_ref=ref.at[my],
                dst_ref=ref.at[my],
                send_sem=send_sems.at[p - 1, t],
                recv_sem=recv_sems.at[p - 1, t],
                device_id=(dst,),
                device_id_type=pl.DeviceIdType.MESH,
            )
            rdma.start()
            rdmas.append(rdma)

    kr_ref[...] = jnp.dot(xbf, wkr_ref[...].astype(jnp.bfloat16),
                          preferred_element_type=jnp.float32)
    qr_full = jnp.dot(xbf, wqr_ref[...].astype(jnp.bfloat16),
                      preferred_element_type=jnp.float32)
    for hh in range(H):
        qr_ref[hh] = qr_full[:, hh * DR:(hh + 1) * DR]

    for rdma in rdmas:
        rdma.wait()


def _mla_body(x_ref, c_ref, wukf_ref, wuvf_ref, kr_ref, qr_h_ref,
              wq_ref, wo_ref, out_ref):
    h = pl.program_id(0)
    scale = (DH + DR) ** -0.5

    q = jnp.dot(x_ref[...], wq_ref[...].astype(jnp.bfloat16),
                preferred_element_type=jnp.float32)
    qr = qr_h_ref[0]

    k = jnp.zeros((BS, DH), jnp.float32)
    v = jnp.zeros((BS, DH), jnp.float32)
    for o in range(N_DEV):
        k = k + jnp.dot(c_ref[o], wukf_ref[o], preferred_element_type=jnp.float32)
        v = v + jnp.dot(c_ref[o], wuvf_ref[o], preferred_element_type=jnp.float32)
    kr = kr_ref[...]

    outs = []
    nt = (((1,), (1,)), ((), ()))
    for b in range(B):
        sl = slice(b * S, (b + 1) * S)
        s = lax.dot_general(q[sl], k[sl], nt, preferred_element_type=jnp.float32)
        s = s + lax.dot_general(qr[sl], kr[sl], nt,
                                preferred_element_type=jnp.float32)
        s = s * scale
        m = jnp.max(s, axis=-1, keepdims=True)
        p = jnp.exp(s - m)
        p = p / jnp.sum(p, axis=-1, keepdims=True)
        outs.append(jnp.dot(p, v[sl], preferred_element_type=jnp.float32))
    o_h = jnp.concatenate(outs, axis=0)

    contrib = jnp.dot(o_h.astype(jnp.bfloat16),
                      wo_ref[...].astype(jnp.bfloat16),
                      preferred_element_type=jnp.float32).reshape(B, S, D)

    @pl.when(h == 0)
    def _():
        out_ref[...] = contrib

    @pl.when(h != 0)
    def _():
        out_ref[...] = out_ref[...] + contrib


def kernel(x, Wdkv, Wuk, Wuv, Wq, Wqr, Wkr, Wo):
    c, wukf, wuvf, kr, qr, xbf = pl.pallas_call(
        _gather_body,
        out_shape=(
            jax.ShapeDtypeStruct((N_DEV, BS, DCS), jnp.bfloat16),
            jax.ShapeDtypeStruct((N_DEV, DCS, D), jnp.bfloat16),
            jax.ShapeDtypeStruct((N_DEV, DCS, D), jnp.bfloat16),
            jax.ShapeDtypeStruct((BS, DR), jnp.float32),
            jax.ShapeDtypeStruct((H, BS, DR), jnp.float32),
            jax.ShapeDtypeStruct((BS, D), jnp.bfloat16),
        ),
        in_specs=[pl.BlockSpec(memory_space=pltpu.VMEM)] * 6,
        out_specs=(pl.BlockSpec(memory_space=pltpu.VMEM),) * 6,
        scratch_shapes=[
            pltpu.SemaphoreType.DMA((N_PEER, N_COMM)),
            pltpu.SemaphoreType.DMA((N_PEER, N_COMM)),
        ],
    )(x, Wdkv, Wuk, Wuv, Wkr, Wqr)

    out = pl.pallas_call(
        _mla_body,
        grid=(H,),
        out_shape=jax.ShapeDtypeStruct((B, S, D), jnp.float32),
        in_specs=[
            pl.BlockSpec((BS, D), lambda h: (0, 0)),
            pl.BlockSpec((N_DEV, BS, DCS), lambda h: (0, 0, 0)),
            pl.BlockSpec((N_DEV, DCS, DH), lambda h: (0, 0, h)),
            pl.BlockSpec((N_DEV, DCS, DH), lambda h: (0, 0, h)),
            pl.BlockSpec((BS, DR), lambda h: (0, 0)),
            pl.BlockSpec((1, BS, DR), lambda h: (h, 0, 0)),
            pl.BlockSpec((D, DH), lambda h: (0, h)),
            pl.BlockSpec((DH, D), lambda h: (h, 0)),
        ],
        out_specs=pl.BlockSpec((B, S, D), lambda h: (0, 0, 0)),
        compiler_params=pltpu.CompilerParams(
            dimension_semantics=("arbitrary",),
        ),
    )(xbf, c, wukf, wuvf, kr, qr, Wq, Wo)
    return out


# device time: 99032 ns/iter; 2.2017x vs baseline; 1.5889x over previous
import jax
import jax.numpy as jnp
from jax import lax
from jax.experimental import pallas as pl
from jax.experimental.pallas import tpu as pltpu

N_DEV = 4
B, S, D = 2, 512, 2048
H, DH, DR = 16, 128, 32
DC = 512
DCS = DC // N_DEV
BS = B * S
N_COMM = 3
N_PEER = N_DEV - 1
NO = 8
DO = D // NO

BF = jnp.bfloat16
F32 = jnp.float32


def _gather_body(x_ref, wdkv_ref, wuk_ref, wuv_ref, wkr_ref, wqr_ref, wq_ref,
                 q_ref, k_ref, v_ref, kr_ref, qr_ref,
                 c_ref, wukf_ref, wuvf_ref, xbf_ref, wqh_ref,
                 send_sems, recv_sems, copy_sem):
    my = lax.axis_index("i")

    for b in range(B):
        xbf_ref[b * S:(b + 1) * S, :] = x_ref[b].astype(BF)
    xbf = xbf_ref[...]
    c_ref[my] = jnp.dot(xbf, wdkv_ref[...].astype(BF),
                        preferred_element_type=F32).astype(BF)
    wukf_ref[my] = wuk_ref[...].astype(BF)
    wuvf_ref[my] = wuv_ref[...].astype(BF)

    rdmas = []
    for p in range(1, N_DEV):
        dst = lax.rem(my + p, N_DEV)
        for t, ref in enumerate((c_ref, wukf_ref, wuvf_ref)):
            rdma = pltpu.make_async_remote_copy(
                src_ref=ref.at[my],
                dst_ref=ref.at[my],
                send_sem=send_sems.at[p - 1, t],
                recv_sem=recv_sems.at[p - 1, t],
                device_id=(dst,),
                device_id_type=pl.DeviceIdType.MESH,
            )
            rdma.start()
            rdmas.append(rdma)

    DQ = D // 2
    for j in range(2):
        cp = pltpu.make_async_copy(
            wq_ref.at[:, pl.ds(j * DQ, DQ)], wqh_ref, copy_sem)
        cp.start()
        cp.wait()
        q_ref[:, j * DQ:(j + 1) * DQ] = jnp.dot(
            xbf, wqh_ref[...].astype(BF),
            preferred_element_type=F32).astype(BF)
    kr_ref[...] = jnp.dot(xbf, wkr_ref[...].astype(BF),
                          preferred_element_type=F32).astype(BF)
    qr_full = jnp.dot(xbf, wqr_ref[...].astype(BF),
                      preferred_element_type=F32).astype(BF)
    for hh in range(H):
        qr_ref[hh] = qr_full[:, hh * DR:(hh + 1) * DR]

    for rdma in rdmas:
        rdma.wait()

    for j in range(2):
        sl = slice(j * DQ, (j + 1) * DQ)
        for src, dst in ((wukf_ref, k_ref), (wuvf_ref, v_ref)):
            acc = jnp.zeros((BS, DQ), F32)
            for o in range(N_DEV):
                acc = acc + jnp.dot(c_ref[o], src[o, :, sl],
                                    preferred_element_type=F32)
            dst[:, sl] = acc.astype(BF)


def _attn_body(q_ref, k_ref, v_ref, kr_ref, qr_h_ref, o_ref):
    scale = (DH + DR) ** -0.5
    qh = jnp.concatenate([q_ref[...], qr_h_ref[0]], axis=1)
    kh = jnp.concatenate([k_ref[...], kr_ref[...]], axis=1)
    nt = (((1,), (1,)), ((), ()))
    for b in range(B):
        sl = slice(b * S, (b + 1) * S)
        s = lax.dot_general(qh[sl], kh[sl], nt, preferred_element_type=F32)
        p = jnp.exp(s * scale)
        denom = jnp.sum(p, axis=-1, keepdims=True)
        o_b = jnp.dot(p.astype(BF), v_ref[sl], preferred_element_type=F32)
        o_ref[sl, :] = (o_b * (1.0 / denom)).astype(BF)


def _proj_body(o_ref, wo_ref, out_ref):
    out_ref[...] = jnp.dot(
        o_ref[...], wo_ref[...].astype(BF),
        preferred_element_type=F32).reshape(B, S, DO)


def kernel(x, Wdkv, Wuk, Wuv, Wq, Wqr, Wkr, Wo):
    q, k, v, kr, qr = pl.pallas_call(
        _gather_body,
        out_shape=(
            jax.ShapeDtypeStruct((BS, D), BF),
            jax.ShapeDtypeStruct((BS, D), BF),
            jax.ShapeDtypeStruct((BS, D), BF),
            jax.ShapeDtypeStruct((BS, DR), BF),
            jax.ShapeDtypeStruct((H, BS, DR), BF),
        ),
        in_specs=[pl.BlockSpec(memory_space=pltpu.VMEM)] * 6
        + [pl.BlockSpec(memory_space=pl.ANY)],
        out_specs=(pl.BlockSpec(memory_space=pltpu.VMEM),) * 5,
        scratch_shapes=[
            pltpu.VMEM((N_DEV, BS, DCS), BF),
            pltpu.VMEM((N_DEV, DCS, D), BF),
            pltpu.VMEM((N_DEV, DCS, D), BF),
            pltpu.VMEM((BS, D), BF),
            pltpu.VMEM((D, D // 2), F32),
            pltpu.SemaphoreType.DMA((N_PEER, N_COMM)),
            pltpu.SemaphoreType.DMA((N_PEER, N_COMM)),
            pltpu.SemaphoreType.DMA,
        ],
    )(x, Wdkv, Wuk, Wuv, Wkr, Wqr, Wq)

    o_all = pl.pallas_call(
        _attn_body,
        grid=(H,),
        out_shape=jax.ShapeDtypeStruct((BS, D), BF),
        in_specs=[
            pl.BlockSpec((BS, DH), lambda h: (0, h)),
            pl.BlockSpec((BS, DH), lambda h: (0, h)),
            pl.BlockSpec((BS, DH), lambda h: (0, h)),
            pl.BlockSpec((BS, DR), lambda h: (0, 0)),
            pl.BlockSpec((1, BS, DR), lambda h: (h, 0, 0)),
        ],
        out_specs=pl.BlockSpec((BS, DH), lambda h: (0, h)),
        compiler_params=pltpu.CompilerParams(
            dimension_semantics=("arbitrary",),
        ),
    )(q, k, v, kr, qr)

    out = pl.pallas_call(
        _proj_body,
        grid=(NO,),
        out_shape=jax.ShapeDtypeStruct((B, S, D), F32),
        in_specs=[
            pl.BlockSpec((BS, D), lambda n: (0, 0)),
            pl.BlockSpec((D, DO), lambda n: (0, n)),
        ],
        out_specs=pl.BlockSpec((B, S, DO), lambda n: (0, 0, n)),
        compiler_params=pltpu.CompilerParams(
            dimension_semantics=("arbitrary",),
        ),
    )(o_all, Wo)
    return out
